# baseline (speedup 1.0000x reference)
import functools

import jax
import jax.numpy as jnp
from jax import lax
from jax.experimental import pallas as pl
from jax.experimental.pallas import tpu as pltpu
from jax.experimental.pallas import tpu_sc as plsc

N_NODES = 10000
NPAD = 10240
ROWS_PER_TILE = NPAD // 16
CHUNK = 64
EBLK = 128
LANES = 16
NBUF = 4
EDGE_ALIGN = 16 * 8 * EBLK


def _ceil_to(x, m):
  return (x + m - 1) // m * m


def _make_sc_aggregate(n_chunks, e_pad):
  chunks_per_core = n_chunks // 2
  blocks_per_tile = e_pad // 16 // EBLK
  mesh = plsc.VectorSubcoreMesh(core_axis_name="c", subcore_axis_name="s")

  @functools.partial(
      pl.kernel,
      out_type=jax.ShapeDtypeStruct((n_chunks * NPAD, CHUNK), jnp.float32),
      mesh=mesh,
      compiler_params=pltpu.CompilerParams(use_tc_tiling_on_sc=False),
      scratch_types=[
          pltpu.VMEM_SHARED((NPAD, CHUNK), jnp.float32),
          pltpu.VMEM_SHARED((NPAD, CHUNK), jnp.float32),
          pltpu.VMEM((blocks_per_tile, EBLK), jnp.int32),
          pltpu.VMEM((EBLK,), jnp.int32),
          pltpu.VMEM((EBLK,), jnp.int32),
          pltpu.VMEM((EBLK,), jnp.int32),
          pltpu.VMEM((EBLK,), jnp.int32),
          pltpu.VMEM((EBLK,), jnp.int32),
          pltpu.VMEM((EBLK,), jnp.int32),
          pltpu.VMEM((EBLK,), jnp.int32),
          pltpu.VMEM((EBLK,), jnp.int32),
          pltpu.VMEM((EBLK, CHUNK), jnp.float32),
          pltpu.VMEM((EBLK, CHUNK), jnp.float32),
          pltpu.VMEM((EBLK, CHUNK), jnp.float32),
          pltpu.VMEM((EBLK, CHUNK), jnp.float32),
          pltpu.VMEM((64, CHUNK), jnp.float32),
          pltpu.SemaphoreType.DMA,
          pltpu.SemaphoreType.DMA,
          pltpu.SemaphoreType.DMA,
          pltpu.SemaphoreType.DMA,
      ],
  )
  def sc_agg(h_cm, sd_hbm, agg_out,
             acc, hch, packed, src0, src1, src2, src3, dst0, dst1, dst2, dst3,
             rows0, rows1, rows2, rows3, zero_v, sem0, sem1, sem2, sem3):
    cid = lax.axis_index("c")
    sid = lax.axis_index("s")
    sems = (sem0, sem1, sem2, sem3)
    rbufs = (rows0, rows1, rows2, rows3)
    sslots = (src0, src1, src2, src3)
    dslots = (dst0, dst1, dst2, dst3)

    @pl.loop(0, 64)
    def _(i):
      for j in range(CHUNK // LANES):
        zero_v[i, pl.ds(j * LANES, LANES)] = jnp.zeros((LANES,), jnp.float32)

    pltpu.sync_copy(sd_hbm.at[pl.ds(sid * blocks_per_tile, blocks_per_tile)],
                    packed)

    def unpack_gather(j, b):
      for t in range(EBLK // LANES):
        s = pl.ds(t * LANES, LANES)
        p = packed[j, s]
        sslots[b][s] = p & 16383
        dslots[b][s] = p >> 14
      pltpu.async_copy(hch.at[sslots[b]], rbufs[b], sems[b])

    def wait_scatter(j, b):
      pltpu.make_async_copy(hch.at[sslots[b]], rbufs[b], sems[b]).wait()
      pltpu.sync_copy(rbufs[b], acc.at[dslots[b]], add=True)

    rslice = pl.ds(sid * ROWS_PER_TILE, ROWS_PER_TILE)
    for k in range(chunks_per_core):
      c = cid * chunks_per_core + k
      crow = pl.multiple_of(c * NPAD + sid * ROWS_PER_TILE, ROWS_PER_TILE)
      cslice = pl.ds(crow, ROWS_PER_TILE)
      pltpu.sync_copy(h_cm.at[cslice], hch.at[rslice])
      for z in range(ROWS_PER_TILE // 64):
        pltpu.sync_copy(
            zero_v, acc.at[pl.ds(sid * ROWS_PER_TILE + z * 64, 64)])
      plsc.subcore_barrier()

      for b in range(NBUF):
        unpack_gather(b, b)

      @pl.loop(0, blocks_per_tile // NBUF - 1)
      def _(i):
        j0 = NBUF * i
        for b in range(NBUF):
          wait_scatter(j0 + b, b)
          unpack_gather(j0 + NBUF + b, b)

      jlast = blocks_per_tile - NBUF
      for b in range(NBUF):
        wait_scatter(jlast + b, b)

      plsc.subcore_barrier()
      pltpu.sync_copy(acc.at[rslice], agg_out.at[cslice])
      plsc.subcore_barrier()

  return sc_agg


def _tc_layer(agg3, W, b, last):
  n_chunks = agg3.shape[0]
  d_in, d_out = W.shape
  nco = d_out // CHUNK
  mb = 1000 if last else 1024
  gm = (N_NODES if last else NPAD) // mb

  def body(x_ref, w_ref, b_ref, o_ref, acc):
    k = pl.program_id(1)

    @pl.when(k == 0)
    def _():
      acc[...] = jnp.zeros_like(acc)

    acc[...] += jnp.dot(x_ref[0], w_ref[...],
                        preferred_element_type=jnp.float32)

    @pl.when(k == n_chunks - 1)
    def _():
      r = jnp.maximum(acc[...] + b_ref[...], 0.0)
      if last:
        o_ref[...] = r
      else:
        for c in range(nco):
          o_ref[c] = r[:, c * CHUNK:(c + 1) * CHUNK]

  if last:
    out_spec = pl.BlockSpec((mb, d_out), lambda i, k: (i, 0))
    out_shape = jax.ShapeDtypeStruct((N_NODES, d_out), jnp.float32)
  else:
    out_spec = pl.BlockSpec((nco, mb, CHUNK), lambda i, k: (0, i, 0))
    out_shape = jax.ShapeDtypeStruct((nco, NPAD, CHUNK), jnp.float32)

  return pl.pallas_call(
      body,
      grid=(gm, n_chunks),
      in_specs=[
          pl.BlockSpec((1, mb, CHUNK), lambda i, k: (k, i, 0)),
          pl.BlockSpec((CHUNK, d_out), lambda i, k: (k, 0)),
          pl.BlockSpec((1, d_out), lambda i, k: (0, 0)),
      ],
      out_specs=out_spec,
      out_shape=out_shape,
      scratch_shapes=[pltpu.VMEM((mb, d_out), jnp.float32)],
  )(agg3, W, b.reshape(1, d_out))


def kernel(x, edge_index, W0, b0, W1, b1, W2, b2):
  src = edge_index[0].astype(jnp.int32)
  dst = edge_index[1].astype(jnp.int32)
  e = src.shape[0]
  e_pad = _ceil_to(e, EDGE_ALIGN)
  pad = e_pad - e
  src_p = jnp.concatenate([src, jnp.zeros((pad,), jnp.int32)])
  dst_p = jnp.concatenate(
      [dst, N_NODES + (jnp.arange(pad, dtype=jnp.int32) % (NPAD - N_NODES))])
  sd_p = (src_p | (dst_p << 14)).reshape(e_pad // EBLK, EBLK)

  d0 = x.shape[1]
  xp = jnp.concatenate([x, jnp.zeros((NPAD - x.shape[0], d0), jnp.float32)])
  h = xp.reshape(NPAD, d0 // CHUNK, CHUNK).transpose(1, 0, 2).reshape(-1, CHUNK)

  layers = ((W0, b0), (W1, b1), (W2, b2))
  for li, (W, b) in enumerate(layers):
    d_in, d_out = W.shape
    n_chunks = d_in // CHUNK
    agg2d = _make_sc_aggregate(n_chunks, e_pad)(h, sd_p)
    out = _tc_layer(agg2d.reshape(n_chunks, NPAD, CHUNK), W, b, last=li == 2)
    if li < 2:
      h = out.reshape(-1, CHUNK)
  return out


# speedup vs baseline: 1.0028x; 1.0028x over previous
import functools

import jax
import jax.numpy as jnp
from jax import lax
from jax.experimental import pallas as pl
from jax.experimental.pallas import tpu as pltpu
from jax.experimental.pallas import tpu_sc as plsc

N_NODES = 10000
NPAD = 10240
ROWS_PER_TILE = NPAD // 16
CHUNK = 64
EBLK = 128
LANES = 16
NBUF = 2
EDGE_ALIGN = 16 * 8 * EBLK


def _ceil_to(x, m):
  return (x + m - 1) // m * m


def _make_sc_aggregate(n_chunks, e_pad):
  chunks_per_core = n_chunks // 2
  blocks_per_tile = e_pad // 16 // EBLK
  mesh = plsc.VectorSubcoreMesh(core_axis_name="c", subcore_axis_name="s")

  @functools.partial(
      pl.kernel,
      out_type=jax.ShapeDtypeStruct((n_chunks * NPAD, CHUNK), jnp.float32),
      mesh=mesh,
      compiler_params=pltpu.CompilerParams(use_tc_tiling_on_sc=False),
      scratch_types=[
          pltpu.VMEM_SHARED((NPAD, CHUNK), jnp.float32),
          pltpu.VMEM_SHARED((NPAD, CHUNK), jnp.float32),
          pltpu.VMEM((blocks_per_tile, EBLK), jnp.int32),
          pltpu.VMEM((EBLK,), jnp.int32),
          pltpu.VMEM((EBLK,), jnp.int32),
          pltpu.VMEM((EBLK,), jnp.int32),
          pltpu.VMEM((EBLK,), jnp.int32),
          pltpu.VMEM((EBLK, CHUNK), jnp.float32),
          pltpu.VMEM((EBLK, CHUNK), jnp.float32),
          pltpu.VMEM((160, CHUNK), jnp.float32),
          pltpu.SemaphoreType.DMA,
          pltpu.SemaphoreType.DMA,
      ],
  )
  def sc_agg(h_cm, sd_hbm, agg_out,
             acc, hch, packed, src0, src1, dst0, dst1, rows0, rows1, zero_v,
             sem0, sem1):
    cid = lax.axis_index("c")
    sid = lax.axis_index("s")
    sems = (sem0, sem1)
    rbufs = (rows0, rows1)
    sslots = (src0, src1)
    dslots = (dst0, dst1)

    @pl.loop(0, 160)
    def _(i):
      for j in range(CHUNK // LANES):
        zero_v[i, pl.ds(j * LANES, LANES)] = jnp.zeros((LANES,), jnp.float32)

    pltpu.sync_copy(sd_hbm.at[pl.ds(sid * blocks_per_tile, blocks_per_tile)],
                    packed)

    def unpack_gather(j, b):
      for t in range(EBLK // LANES):
        s = pl.ds(t * LANES, LANES)
        p = packed[j, s]
        sslots[b][s] = p & 16383
        dslots[b][s] = p >> 14
      pltpu.async_copy(hch.at[sslots[b]], rbufs[b], sems[b])

    def wait_scatter(j, b):
      pltpu.make_async_copy(hch.at[sslots[b]], rbufs[b], sems[b]).wait()
      pltpu.sync_copy(rbufs[b], acc.at[dslots[b]], add=True)

    rslice = pl.ds(sid * ROWS_PER_TILE, ROWS_PER_TILE)
    for k in range(chunks_per_core):
      c = cid * chunks_per_core + k
      crow = pl.multiple_of(c * NPAD + sid * ROWS_PER_TILE, ROWS_PER_TILE)
      cslice = pl.ds(crow, ROWS_PER_TILE)
      pltpu.sync_copy(h_cm.at[cslice], hch.at[rslice])
      for z in range(ROWS_PER_TILE // 160):
        pltpu.sync_copy(
            zero_v, acc.at[pl.ds(sid * ROWS_PER_TILE + z * 160, 160)])
      plsc.subcore_barrier()

      for b in range(NBUF):
        unpack_gather(b, b)

      @pl.loop(0, blocks_per_tile // NBUF - 1)
      def _(i):
        j0 = NBUF * i
        for b in range(NBUF):
          wait_scatter(j0 + b, b)
          unpack_gather(j0 + NBUF + b, b)

      jlast = blocks_per_tile - NBUF
      for b in range(NBUF):
        wait_scatter(jlast + b, b)

      plsc.subcore_barrier()
      pltpu.sync_copy(acc.at[rslice], agg_out.at[cslice])
      plsc.subcore_barrier()

  return sc_agg


def _tc_layer(agg3, W, b, last):
  n_chunks = agg3.shape[0]
  d_in, d_out = W.shape
  nco = d_out // CHUNK
  mb = 1000 if last else 1024
  gm = (N_NODES if last else NPAD) // mb

  def body(x_ref, w_ref, b_ref, o_ref, acc):
    k = pl.program_id(1)

    @pl.when(k == 0)
    def _():
      acc[...] = jnp.zeros_like(acc)

    acc[...] += jnp.dot(x_ref[0], w_ref[...],
                        preferred_element_type=jnp.float32)

    @pl.when(k == n_chunks - 1)
    def _():
      r = jnp.maximum(acc[...] + b_ref[...], 0.0)
      if last:
        o_ref[...] = r
      else:
        for c in range(nco):
          o_ref[c] = r[:, c * CHUNK:(c + 1) * CHUNK]

  if last:
    out_spec = pl.BlockSpec((mb, d_out), lambda i, k: (i, 0))
    out_shape = jax.ShapeDtypeStruct((N_NODES, d_out), jnp.float32)
  else:
    out_spec = pl.BlockSpec((nco, mb, CHUNK), lambda i, k: (0, i, 0))
    out_shape = jax.ShapeDtypeStruct((nco, NPAD, CHUNK), jnp.float32)

  return pl.pallas_call(
      body,
      grid=(gm, n_chunks),
      in_specs=[
          pl.BlockSpec((1, mb, CHUNK), lambda i, k: (k, i, 0)),
          pl.BlockSpec((CHUNK, d_out), lambda i, k: (k, 0)),
          pl.BlockSpec((1, d_out), lambda i, k: (0, 0)),
      ],
      out_specs=out_spec,
      out_shape=out_shape,
      scratch_shapes=[pltpu.VMEM((mb, d_out), jnp.float32)],
  )(agg3, W, b.reshape(1, d_out))


def kernel(x, edge_index, W0, b0, W1, b1, W2, b2):
  src = edge_index[0].astype(jnp.int32)
  dst = edge_index[1].astype(jnp.int32)
  e = src.shape[0]
  e_pad = _ceil_to(e, EDGE_ALIGN)
  pad = e_pad - e
  src_p = jnp.concatenate([src, jnp.zeros((pad,), jnp.int32)])
  dst_p = jnp.concatenate(
      [dst, N_NODES + (jnp.arange(pad, dtype=jnp.int32) % (NPAD - N_NODES))])
  sd_p = (src_p | (dst_p << 14)).reshape(e_pad // EBLK, EBLK)

  d0 = x.shape[1]
  xp = jnp.concatenate([x, jnp.zeros((NPAD - x.shape[0], d0), jnp.float32)])
  h = xp.reshape(NPAD, d0 // CHUNK, CHUNK).transpose(1, 0, 2).reshape(-1, CHUNK)

  layers = ((W0, b0), (W1, b1), (W2, b2))
  for li, (W, b) in enumerate(layers):
    d_in, d_out = W.shape
    n_chunks = d_in // CHUNK
    agg2d = _make_sc_aggregate(n_chunks, e_pad)(h, sd_p)
    out = _tc_layer(agg2d.reshape(n_chunks, NPAD, CHUNK), W, b, last=li == 2)
    if li < 2:
      h = out.reshape(-1, CHUNK)
  return out
